# Initial kernel scaffold; baseline (speedup 1.0000x reference)
import functools

import jax
import jax.numpy as jnp
from jax import lax
from jax.experimental import pallas as pl
from jax.experimental.pallas import tpu as pltpu
from jax.experimental.pallas import tpu_sc as plsc

EMBED_DIM = 64

_info = plsc.get_sparse_core_info()
_NC, _NS = _info.num_cores, _info.num_subcores
_NW = _NC * _NS

_CHUNK = 128


def _embed_kernel(n_rows: int):
  b_per_w = n_rows // _NW
  n_chunks = b_per_w // _CHUNK
  mesh = plsc.VectorSubcoreMesh(core_axis_name="c", subcore_axis_name="s")

  @functools.partial(
      pl.kernel,
      out_type=jax.ShapeDtypeStruct((n_rows, EMBED_DIM), jnp.float32),
      mesh=mesh,
      scratch_types=[
          pltpu.VMEM((_CHUNK,), jnp.int32),
          pltpu.VMEM((_CHUNK, EMBED_DIM), jnp.float32),
          pltpu.SemaphoreType.DMA,
      ],
  )
  def body(idx_hbm, table_hbm, out_hbm, idx_v, rows_v, sem):
    wid = lax.axis_index("s") * _NC + lax.axis_index("c")
    base = wid * b_per_w

    def step(i, carry):
      off = base + i * _CHUNK
      pltpu.sync_copy(idx_hbm.at[pl.ds(off, _CHUNK)], idx_v)
      pltpu.async_copy(table_hbm.at[idx_v], rows_v, sem).wait()
      pltpu.sync_copy(rows_v, out_hbm.at[pl.ds(off, _CHUNK)])
      return carry

    lax.fori_loop(0, n_chunks, step, 0)

  return body


def kernel(x, table):
  n_rows = x.shape[0] * x.shape[1]
  flat = x.reshape(n_rows).astype(jnp.int32)
  out = _embed_kernel(n_rows)(flat, table)
  return out.reshape(x.shape[0], x.shape[1], EMBED_DIM)


# speedup vs baseline: 3.1775x; 3.1775x over previous
import functools

import jax
import jax.numpy as jnp
from jax import lax
from jax.experimental import pallas as pl
from jax.experimental.pallas import tpu as pltpu
from jax.experimental.pallas import tpu_sc as plsc

EMBED_DIM = 64

_info = plsc.get_sparse_core_info()
_NC, _NS = _info.num_cores, _info.num_subcores
_NW = _NC * _NS

_CHUNK = 128


def _embed_kernel(n_rows: int):
  b_per_w = n_rows // _NW
  n_chunks = b_per_w // _CHUNK
  mesh = plsc.VectorSubcoreMesh(core_axis_name="c", subcore_axis_name="s")

  @functools.partial(
      pl.kernel,
      out_type=jax.ShapeDtypeStruct((n_rows, EMBED_DIM), jnp.float32),
      mesh=mesh,
      scratch_types=[
          pltpu.VMEM((_CHUNK,), jnp.int32),
          pltpu.VMEM((_CHUNK, EMBED_DIM), jnp.float32),
          pltpu.SemaphoreType.DMA,
      ],
      compiler_params=pltpu.CompilerParams(use_tc_tiling_on_sc=False),
  )
  def body(idx_hbm, table_hbm, out_hbm, idx_v, rows_v, sem):
    wid = lax.axis_index("s") * _NC + lax.axis_index("c")
    base = wid * b_per_w

    def step(i, carry):
      off = base + i * _CHUNK
      pltpu.sync_copy(idx_hbm.at[pl.ds(off, _CHUNK)], idx_v)
      pltpu.async_copy(table_hbm.at[idx_v], rows_v, sem).wait()
      pltpu.sync_copy(rows_v, out_hbm.at[pl.ds(off, _CHUNK)])
      return carry

    lax.fori_loop(0, n_chunks, step, 0)

  return body


def kernel(x, table):
  n_rows = x.shape[0] * x.shape[1]
  flat = x.reshape(n_rows).astype(jnp.int32)
  out = _embed_kernel(n_rows)(flat, table)
  return out.reshape(x.shape[0], x.shape[1], EMBED_DIM)


# speedup vs baseline: 4.2476x; 1.3368x over previous
import functools

import jax
import jax.numpy as jnp
from jax import lax
from jax.experimental import pallas as pl
from jax.experimental.pallas import tpu as pltpu
from jax.experimental.pallas import tpu_sc as plsc

EMBED_DIM = 64

_info = plsc.get_sparse_core_info()
_NC, _NS = _info.num_cores, _info.num_subcores
_NW = _NC * _NS

_CHUNK = 128
_K = 5
_GROUP = _K * _CHUNK
_NBUF = 2


def _embed_kernel(n_rows: int):
  b_per_w = n_rows // _NW
  n_chunks = b_per_w // _CHUNK
  n_groups = b_per_w // _GROUP
  mesh = plsc.VectorSubcoreMesh(core_axis_name="c", subcore_axis_name="s")

  @functools.partial(
      pl.kernel,
      out_type=jax.ShapeDtypeStruct((n_rows, EMBED_DIM), jnp.float32),
      mesh=mesh,
      scratch_types=[
          pltpu.VMEM((n_chunks, _CHUNK), jnp.int32),
          pltpu.VMEM((_GROUP, EMBED_DIM), jnp.float32),
          pltpu.VMEM((_GROUP, EMBED_DIM), jnp.float32),
          pltpu.SemaphoreType.DMA,
          pltpu.SemaphoreType.DMA,
          pltpu.SemaphoreType.DMA,
          pltpu.SemaphoreType.DMA,
      ],
      compiler_params=pltpu.CompilerParams(use_tc_tiling_on_sc=False),
  )
  def body(idx_hbm, table_hbm, out_hbm, idx_v, rows_a, rows_b,
           sem_ga, sem_gb, sem_sa, sem_sb):
    wid = lax.axis_index("s") * _NC + lax.axis_index("c")
    base = wid * b_per_w
    rows = (rows_a, rows_b)
    sem_g = (sem_ga, sem_gb)
    sem_s = (sem_sa, sem_sb)

    pltpu.sync_copy(idx_hbm.at[pl.ds(wid * n_chunks, n_chunks)], idx_v)

    def fire_gathers(g, b):
      hs = []
      for j in range(_K):
        c = g * _K + j
        hs.append(pltpu.async_copy(
            table_hbm.at[idx_v.at[c]],
            rows[b].at[pl.ds(j * _CHUNK, _CHUNK)],
            sem_g[b]))
      return hs

    def drain(hs):
      for h in hs:
        h.wait()

    def fire_store(g, b):
      pltpu.async_copy(rows[b], out_hbm.at[pl.ds(base + g * _GROUP, _GROUP)],
                       sem_s[b])

    def wait_store(b):
      pltpu.make_async_copy(rows[b], out_hbm.at[pl.ds(base, _GROUP)],
                            sem_s[b]).wait()

    for b in range(_NBUF):
      drain(fire_gathers(b, b))
      fire_store(b, b)

    def step(i, carry):
      for b in range(_NBUF):
        g = i * _NBUF + b
        wait_store(b)
        drain(fire_gathers(g, b))
        fire_store(g, b)
      return carry

    lax.fori_loop(1, n_groups // _NBUF, step, 0)

    for b in range(_NBUF):
      wait_store(b)

  return body


def kernel(x, table):
  n_rows = x.shape[0] * x.shape[1]
  flat = x.reshape(n_rows // _CHUNK, _CHUNK).astype(jnp.int32)
  out = _embed_kernel(n_rows)(flat, table)
  return out.reshape(x.shape[0], x.shape[1], EMBED_DIM)
